# baseline (speedup 1.0000x reference)
import functools

import jax
import jax.numpy as jnp
from jax import lax
from jax.experimental import pallas as pl
from jax.experimental.pallas import tpu as pltpu
from jax.experimental.pallas import tpu_sc as plsc

ALPHA = 0.1
STEPS = 4

B, L, D = 8, 4096, 1024
LANES = 16
NC, NS = 2, 16
NW = NC * NS
STRIPS = B * (D // LANES)
STRIPS_PER_W = STRIPS // NW
DCHUNKS = D // LANES


def _sc_body(x_hbm, o_hbm, buf, sem):
    wid = lax.axis_index("s") * NC + lax.axis_index("c")
    for k in range(STRIPS_PER_W):
        s = wid * STRIPS_PER_W + k
        b = s // DCHUNKS
        d0 = (s % DCHUNKS) * LANES
        pltpu.async_copy(x_hbm.at[b, :, pl.ds(d0, LANES)], buf, sem).wait()

        for _ in range(STEPS):
            def step_row(i, carry):
                prev, cur = carry
                nxt = buf[i + 1]
                lap = nxt - 2.0 * cur + prev
                buf[i] = cur + ALPHA * lap
                return cur, nxt

            lax.fori_loop(1, L - 1, step_row, (buf[0], buf[1]))

        pltpu.async_copy(buf, o_hbm.at[b, :, pl.ds(d0, LANES)], sem).wait()


@jax.jit
def kernel(x):
    mesh = plsc.VectorSubcoreMesh(core_axis_name="c", subcore_axis_name="s")
    return pl.kernel(
        _sc_body,
        out_type=jax.ShapeDtypeStruct((B, L, D), jnp.float32),
        mesh=mesh,
        scratch_types=[
            pltpu.VMEM((L, LANES), jnp.float32),
            pltpu.SemaphoreType.DMA,
        ],
        compiler_params=pltpu.CompilerParams(use_tc_tiling_on_sc=False),
    )(x)


# speedup vs baseline: 4.8422x; 4.8422x over previous
import jax
import jax.numpy as jnp
import numpy as np
from jax import lax
from jax.experimental import pallas as pl
from jax.experimental.pallas import tpu as pltpu
from jax.experimental.pallas import tpu_sc as plsc

ALPHA = 0.1
STEPS = 4

LANES = 16
NC, NS = 2, 16
NW = NC * NS
UNROLL = 8

_taps = np.array([ALPHA, 1.0 - 2.0 * ALPHA, ALPHA], dtype=np.float64)
_k = np.array([1.0])
for _ in range(STEPS):
    _k = np.convolve(_k, _taps)
D0, D1, D2, D3, D4 = (float(_k[STEPS + j]) for j in range(STEPS + 1))


def _edge_steps(rows):
    h = list(rows)
    for _ in range(STEPS):
        upd = [h[j] + ALPHA * (h[j + 1] - 2.0 * h[j] + h[j - 1])
               for j in range(1, 7)]
        h[1:7] = upd
    return h


def _strip_task(buf, L):
    w = tuple(buf[j] for j in range(8))

    h = _edge_steps(w)
    buf[1], buf[2], buf[3] = h[1], h[2], h[3]

    n_groups = (L - 8) // UNROLL

    def group(t, w):
        base = 4 + t * UNROLL
        for u in range(UNROLL):
            w8 = buf[base + u + 4]
            out = (D0 * w[4] + D1 * (w[3] + w[5]) + D2 * (w[2] + w[6])
                   + D3 * (w[1] + w[7]) + D4 * (w[0] + w8))
            buf[base + u] = out
            w = w[1:] + (w8,)
        return w

    w = lax.fori_loop(0, n_groups, group, w)

    t = _edge_steps(w)
    buf[L - 4], buf[L - 3], buf[L - 2] = t[4], t[5], t[6]


def _sc_body(x_hbm, o_hbm, buf, sem, *, B, L, D):
    dchunks = D // LANES
    strips_per_w = (B * dchunks) // NW
    wid = lax.axis_index("s") * NC + lax.axis_index("c")

    def strip(k, carry):
        s = wid * strips_per_w + k
        b = s // dchunks
        d0 = (s % dchunks) * LANES
        pltpu.async_copy(x_hbm.at[b, :, pl.ds(d0, LANES)], buf, sem).wait()
        _strip_task(buf, L)
        pltpu.async_copy(buf, o_hbm.at[b, :, pl.ds(d0, LANES)], sem).wait()
        return carry

    lax.fori_loop(0, strips_per_w, strip, 0)


@jax.jit
def kernel(x):
    B, L, D = x.shape
    assert D % LANES == 0 and (B * (D // LANES)) % NW == 0
    assert (L - 8) % UNROLL == 0

    import functools
    body = functools.partial(_sc_body, B=B, L=L, D=D)
    mesh = plsc.VectorSubcoreMesh(core_axis_name="c", subcore_axis_name="s")
    return pl.kernel(
        body,
        out_type=jax.ShapeDtypeStruct((B, L, D), jnp.float32),
        mesh=mesh,
        scratch_types=[
            pltpu.VMEM((L, LANES), jnp.float32),
            pltpu.SemaphoreType.DMA,
        ],
        compiler_params=pltpu.CompilerParams(use_tc_tiling_on_sc=False),
    )(x)


# speedup vs baseline: 5.0719x; 1.0474x over previous
import jax
import jax.numpy as jnp
import numpy as np
from jax import lax
from jax.experimental import pallas as pl
from jax.experimental.pallas import tpu as pltpu
from jax.experimental.pallas import tpu_sc as plsc

ALPHA = 0.1
STEPS = 4

LANES = 16
NC, NS = 2, 16
NW = NC * NS
UNROLL = 8

_taps = np.array([ALPHA, 1.0 - 2.0 * ALPHA, ALPHA], dtype=np.float64)
_k = np.array([1.0])
for _ in range(STEPS):
    _k = np.convolve(_k, _taps)
D0, D1, D2, D3, D4 = (float(_k[STEPS + j]) for j in range(STEPS + 1))


def _edge_steps(rows):
    h = list(rows)
    for _ in range(STEPS):
        upd = [h[j] + ALPHA * (h[j + 1] - 2.0 * h[j] + h[j - 1])
               for j in range(1, 7)]
        h[1:7] = upd
    return h


def _conv_pass(in_buf, out_buf, L):
    w = tuple(in_buf[j] for j in range(8))

    h = _edge_steps(w)
    in_buf[1], in_buf[2], in_buf[3] = h[1], h[2], h[3]

    n_groups = (L - 8) // UNROLL

    def group(t, w):
        base = 4 + t * UNROLL
        for u in range(UNROLL):
            w8 = in_buf[base + u + 4]
            out = (D0 * w[4] + D1 * (w[3] + w[5]) + D2 * (w[2] + w[6])
                   + D3 * (w[1] + w[7]) + D4 * (w[0] + w8))
            out_buf[base + u - 4] = out
            w = w[1:] + (w8,)
        return w

    w = lax.fori_loop(0, n_groups, group, w)

    t = _edge_steps(w)
    in_buf[L - 4], in_buf[L - 3], in_buf[L - 2] = t[4], t[5], t[6]


def _sc_body(x_hbm, o_hbm, in_buf, out_buf, si, so_big, so_e, *, B, L, D):
    dchunks = D // LANES
    strips_per_w = (B * dchunks) // NW
    wid = lax.axis_index("s") * NC + lax.axis_index("c")

    def hbm_in(s):
        b, d0 = s // dchunks, (s % dchunks) * LANES
        return x_hbm.at[b, :, pl.ds(d0, LANES)]

    def hbm_out(s, r0, n):
        b, d0 = s // dchunks, (s % dchunks) * LANES
        return o_hbm.at[b, pl.ds(r0, n), pl.ds(d0, LANES)]

    first = wid * strips_per_w
    pltpu.async_copy(hbm_in(first), in_buf, si)

    def strip(k, carry):
        s = wid * strips_per_w + k
        pltpu.make_async_copy(hbm_in(s), in_buf, si).wait()
        @pl.when(k > 0)
        def _():
            pltpu.make_async_copy(out_buf, hbm_out(s, 4, L - 8), so_big).wait()

        _conv_pass(in_buf, out_buf, L)

        pltpu.async_copy(in_buf.at[pl.ds(0, 4)], hbm_out(s, 0, 4), so_e)
        pltpu.async_copy(in_buf.at[pl.ds(L - 4, 4)], hbm_out(s, L - 4, 4), so_e)
        pltpu.async_copy(out_buf, hbm_out(s, 4, L - 8), so_big)
        pltpu.make_async_copy(in_buf.at[pl.ds(0, 4)], hbm_out(s, 0, 4), so_e).wait()
        pltpu.make_async_copy(in_buf.at[pl.ds(L - 4, 4)], hbm_out(s, L - 4, 4), so_e).wait()

        @pl.when(k + 1 < strips_per_w)
        def _():
            pltpu.async_copy(hbm_in(s + 1), in_buf, si)

        return carry

    lax.fori_loop(0, strips_per_w, strip, 0)
    last = wid * strips_per_w + strips_per_w - 1
    pltpu.make_async_copy(out_buf, hbm_out(last, 4, L - 8), so_big).wait()


@jax.jit
def kernel(x):
    B, L, D = x.shape
    assert D % LANES == 0 and (B * (D // LANES)) % NW == 0
    assert (L - 8) % UNROLL == 0

    import functools
    body = functools.partial(_sc_body, B=B, L=L, D=D)
    mesh = plsc.VectorSubcoreMesh(core_axis_name="c", subcore_axis_name="s")
    return pl.kernel(
        body,
        out_type=jax.ShapeDtypeStruct((B, L, D), jnp.float32),
        mesh=mesh,
        scratch_types=[
            pltpu.VMEM((L, LANES), jnp.float32),
            pltpu.VMEM((L - 8, LANES), jnp.float32),
            pltpu.SemaphoreType.DMA,
            pltpu.SemaphoreType.DMA,
            pltpu.SemaphoreType.DMA,
        ],
        compiler_params=pltpu.CompilerParams(use_tc_tiling_on_sc=False),
    )(x)


# speedup vs baseline: 5.3839x; 1.0615x over previous
import jax
import jax.numpy as jnp
import numpy as np
from jax import lax
from jax.experimental import pallas as pl
from jax.experimental.pallas import tpu as pltpu
from jax.experimental.pallas import tpu_sc as plsc

ALPHA = 0.1
STEPS = 4

LANES = 16
NC, NS = 2, 16
NW = NC * NS
UNROLL = 14
NSEG = 4

_taps = np.array([ALPHA, 1.0 - 2.0 * ALPHA, ALPHA], dtype=np.float64)
_k = np.array([1.0])
for _ in range(STEPS):
    _k = np.convolve(_k, _taps)
D0, D1, D2, D3, D4 = (float(_k[STEPS + j]) for j in range(STEPS + 1))


def _edge_steps(rows):
    h = list(rows)
    for _ in range(STEPS):
        upd = [h[j] + ALPHA * (h[j + 1] - 2.0 * h[j] + h[j - 1])
               for j in range(1, 7)]
        h[1:7] = upd
    return h


def _sc_body(x_hbm, o_hbm, in_buf, out_h0, out_h1, si, so0, so1, so_e,
             *, B, L, D):
    dchunks = D // LANES
    strips_per_w = (B * dchunks) // NW
    wid = lax.axis_index("s") * NC + lax.axis_index("c")
    seg = (L - 8) // NSEG
    groups = seg // UNROLL
    halves = (out_h0, out_h1)
    sems = (so0, so1)

    def hbm_in(s):
        b, d0 = s // dchunks, (s % dchunks) * LANES
        return x_hbm.at[b, :, pl.ds(d0, LANES)]

    def hbm_out(s, r0, n):
        b, d0 = s // dchunks, (s % dchunks) * LANES
        return o_hbm.at[b, pl.ds(r0, n), pl.ds(d0, LANES)]

    first = wid * strips_per_w
    pltpu.async_copy(hbm_in(first), in_buf, si)

    def strip(k, carry):
        s = wid * strips_per_w + k
        pltpu.make_async_copy(hbm_in(s), in_buf, si).wait()

        w = tuple(in_buf[j] for j in range(8))
        h = _edge_steps(w)
        in_buf[1], in_buf[2], in_buf[3] = h[1], h[2], h[3]
        pltpu.async_copy(in_buf.at[pl.ds(0, 4)], hbm_out(s, 0, 4), so_e)

        for sg in range(NSEG):
            half, sem = halves[sg % 2], sems[sg % 2]
            if sg >= 2:
                pltpu.make_async_copy(half, hbm_out(s, 4 + sg * seg, seg),
                                      sem).wait()
            else:
                @pl.when(k > 0)
                def _():
                    pltpu.make_async_copy(half, hbm_out(s, 4 + sg * seg, seg),
                                          sem).wait()

            def group(t, w, sg=sg, half=half):
                base = 4 + sg * seg + t * UNROLL
                for u in range(UNROLL):
                    w8 = in_buf[base + u + 4]
                    out = (D0 * w[4] + D1 * (w[3] + w[5]) + D2 * (w[2] + w[6])
                           + D3 * (w[1] + w[7]) + D4 * (w[0] + w8))
                    half[t * UNROLL + u] = out
                    w = w[1:] + (w8,)
                return w

            w = lax.fori_loop(0, groups, group, w)
            pltpu.async_copy(half, hbm_out(s, 4 + sg * seg, seg), sem)

        t = _edge_steps(w)
        in_buf[L - 4], in_buf[L - 3], in_buf[L - 2] = t[4], t[5], t[6]
        pltpu.async_copy(in_buf.at[pl.ds(L - 4, 4)], hbm_out(s, L - 4, 4), so_e)
        pltpu.make_async_copy(in_buf.at[pl.ds(0, 4)], hbm_out(s, 0, 4), so_e).wait()
        pltpu.make_async_copy(in_buf.at[pl.ds(L - 4, 4)], hbm_out(s, L - 4, 4), so_e).wait()

        @pl.when(k + 1 < strips_per_w)
        def _():
            pltpu.async_copy(hbm_in(s + 1), in_buf, si)

        return carry

    lax.fori_loop(0, strips_per_w, strip, 0)
    last = wid * strips_per_w + strips_per_w - 1
    for sg in (NSEG - 2, NSEG - 1):
        pltpu.make_async_copy(halves[sg % 2], hbm_out(last, 4 + sg * seg, seg),
                              sems[sg % 2]).wait()


@jax.jit
def kernel(x):
    B, L, D = x.shape
    assert D % LANES == 0 and (B * (D // LANES)) % NW == 0
    assert (L - 8) % NSEG == 0 and ((L - 8) // NSEG) % UNROLL == 0

    import functools
    body = functools.partial(_sc_body, B=B, L=L, D=D)
    mesh = plsc.VectorSubcoreMesh(core_axis_name="c", subcore_axis_name="s")
    return pl.kernel(
        body,
        out_type=jax.ShapeDtypeStruct((B, L, D), jnp.float32),
        mesh=mesh,
        scratch_types=[
            pltpu.VMEM((L, LANES), jnp.float32),
            pltpu.VMEM(((L - 8) // NSEG, LANES), jnp.float32),
            pltpu.VMEM(((L - 8) // NSEG, LANES), jnp.float32),
            pltpu.SemaphoreType.DMA,
            pltpu.SemaphoreType.DMA,
            pltpu.SemaphoreType.DMA,
            pltpu.SemaphoreType.DMA,
        ],
        compiler_params=pltpu.CompilerParams(use_tc_tiling_on_sc=False),
    )(x)


# speedup vs baseline: 7.4679x; 1.3871x over previous
import jax
import jax.numpy as jnp
import numpy as np
from jax import lax
from jax.experimental import pallas as pl
from jax.experimental.pallas import tpu as pltpu
from jax.experimental.pallas import tpu_sc as plsc

ALPHA = 0.1
STEPS = 4

LANES = 16
NC, NS = 2, 16
NW = NC * NS
UNROLL = 14
NSEG = 4

_taps = np.array([ALPHA, 1.0 - 2.0 * ALPHA, ALPHA], dtype=np.float64)
_k = np.array([1.0])
for _ in range(STEPS):
    _k = np.convolve(_k, _taps)
D0, D1, D2, D3, D4 = (float(_k[STEPS + j]) for j in range(STEPS + 1))


def _edge_steps(rows):
    h = list(rows)
    for _ in range(STEPS):
        upd = [h[j] + ALPHA * (h[j + 1] - 2.0 * h[j] + h[j - 1])
               for j in range(1, 7)]
        h[1:7] = upd
    return h


def _sc_body(x_hbm, o_hbm, in_buf, out_h0, out_h1, si, so0, so1, so_e,
             *, B, L, D):
    dchunks = D // LANES
    strips_per_w = (B * dchunks) // NW
    wid = lax.axis_index("s") * NC + lax.axis_index("c")
    seg = (L - 8) // NSEG
    groups = seg // UNROLL
    halves = (out_h0, out_h1)
    sems = (so0, so1)

    def hbm_in(s):
        b, d0 = s // dchunks, (s % dchunks) * LANES
        return x_hbm.at[b, :, pl.ds(d0, LANES)]

    def hbm_out(s, r0, n):
        b, d0 = s // dchunks, (s % dchunks) * LANES
        return o_hbm.at[b, pl.ds(r0, n), pl.ds(d0, LANES)]

    first = wid * strips_per_w
    pltpu.async_copy(hbm_in(first), in_buf, si)

    def strip(k, carry):
        s = wid * strips_per_w + k
        pltpu.make_async_copy(hbm_in(s), in_buf, si).wait()

        w = tuple(in_buf[j] for j in range(8))
        h = _edge_steps(w)
        in_buf[1], in_buf[2], in_buf[3] = h[1], h[2], h[3]
        pltpu.async_copy(in_buf.at[pl.ds(0, 4)], hbm_out(s, 0, 4), so_e)

        for sg in range(NSEG):
            half, sem = halves[sg % 2], sems[sg % 2]
            if sg >= 2:
                pltpu.make_async_copy(half, hbm_out(s, 4 + sg * seg, seg),
                                      sem).wait()
            else:
                @pl.when(k > 0)
                def _():
                    pltpu.make_async_copy(half, hbm_out(s, 4 + sg * seg, seg),
                                          sem).wait()

            def group(t, w, sg=sg, half=half):
                base = 4 + sg * seg + t * UNROLL
                for u in range(UNROLL):
                    w8 = in_buf[base + u + 4]
                    out = (D0 * w[4] + D1 * (w[3] + w[5]) + D2 * (w[2] + w[6])
                           + D3 * (w[1] + w[7]) + D4 * (w[0] + w8))
                    half[t * UNROLL + u] = out
                    w = w[1:] + (w8,)
                return w

            w = lax.fori_loop(0, groups, group, w)
            pltpu.async_copy(half, hbm_out(s, 4 + sg * seg, seg), sem)

        t = _edge_steps(w)
        in_buf[L - 4], in_buf[L - 3], in_buf[L - 2] = t[4], t[5], t[6]
        pltpu.async_copy(in_buf.at[pl.ds(L - 4, 4)], hbm_out(s, L - 4, 4), so_e)
        pltpu.make_async_copy(in_buf.at[pl.ds(0, 4)], hbm_out(s, 0, 4), so_e).wait()
        pltpu.make_async_copy(in_buf.at[pl.ds(L - 4, 4)], hbm_out(s, L - 4, 4), so_e).wait()

        @pl.when(k + 1 < strips_per_w)
        def _():
            pltpu.async_copy(hbm_in(s + 1), in_buf, si)

        return carry

    lax.fori_loop(0, strips_per_w, strip, 0)
    last = wid * strips_per_w + strips_per_w - 1
    for sg in (NSEG - 2, NSEG - 1):
        pltpu.make_async_copy(halves[sg % 2], hbm_out(last, 4 + sg * seg, seg),
                              sems[sg % 2]).wait()


def _sc_mixer(x):
    B, L, D = x.shape
    assert D % LANES == 0 and (B * (D // LANES)) % NW == 0
    assert (L - 8) % NSEG == 0 and ((L - 8) // NSEG) % UNROLL == 0

    import functools
    body = functools.partial(_sc_body, B=B, L=L, D=D)
    mesh = plsc.VectorSubcoreMesh(core_axis_name="c", subcore_axis_name="s")
    return pl.kernel(
        body,
        out_type=jax.ShapeDtypeStruct((B, L, D), jnp.float32),
        mesh=mesh,
        scratch_types=[
            pltpu.VMEM((L, LANES), jnp.float32),
            pltpu.VMEM(((L - 8) // NSEG, LANES), jnp.float32),
            pltpu.VMEM(((L - 8) // NSEG, LANES), jnp.float32),
            pltpu.SemaphoreType.DMA,
            pltpu.SemaphoreType.DMA,
            pltpu.SemaphoreType.DMA,
            pltpu.SemaphoreType.DMA,
        ],
        compiler_params=pltpu.CompilerParams(use_tc_tiling_on_sc=False),
    )(x)


def _tc_block(x_ref, o_ref):
    y = x_ref[0]
    L = y.shape[0]

    def edges(h):
        for _ in range(STEPS):
            upd = h[1:7] + ALPHA * (h[2:8] - 2.0 * h[1:7] + h[0:6])
            h = jnp.concatenate([h[:1], upd, h[7:]], axis=0)
        return h

    h = edges(y[0:8])
    t = edges(y[L - 8:L])
    mid = (D0 * y[4:-4] + D1 * (y[3:-5] + y[5:-3]) + D2 * (y[2:-6] + y[6:-2])
           + D3 * (y[1:-7] + y[7:-1]) + D4 * (y[:-8] + y[8:]))
    o_ref[0] = jnp.concatenate(
        [y[:1], h[1:4], mid, t[4:7], y[-1:]], axis=0)


def _tc_mixer(x):
    B, L, D = x.shape
    W = 128
    return pl.pallas_call(
        _tc_block,
        grid=(B, D // W),
        in_specs=[pl.BlockSpec((1, L, W), lambda i, j: (i, 0, j))],
        out_specs=pl.BlockSpec((1, L, W), lambda i, j: (i, 0, j)),
        out_shape=jax.ShapeDtypeStruct((B, L, D), jnp.float32),
    )(x)


SC_BATCHES = 2


@jax.jit
def kernel(x):
    B = x.shape[0]
    sc_out = _sc_mixer(x[:SC_BATCHES])
    tc_out = _tc_mixer(x[SC_BATCHES:]) if SC_BATCHES < B else None
    if tc_out is None:
        return sc_out
    return jnp.concatenate([sc_out, tc_out], axis=0)


# speedup vs baseline: 19.6328x; 2.6289x over previous
import jax
import jax.numpy as jnp
import numpy as np
from jax import lax
from jax.experimental import pallas as pl
from jax.experimental.pallas import tpu as pltpu
from jax.experimental.pallas import tpu_sc as plsc

ALPHA = 0.1
STEPS = 4

LANES = 16
NC, NS = 2, 16
NW = NC * NS
UNROLL = 14
NSEG = 4

_taps = np.array([ALPHA, 1.0 - 2.0 * ALPHA, ALPHA], dtype=np.float64)
_k = np.array([1.0])
for _ in range(STEPS):
    _k = np.convolve(_k, _taps)
D0, D1, D2, D3, D4 = (float(_k[STEPS + j]) for j in range(STEPS + 1))


def _edge_steps(rows):
    h = list(rows)
    for _ in range(STEPS):
        upd = [h[j] + ALPHA * (h[j + 1] - 2.0 * h[j] + h[j - 1])
               for j in range(1, 7)]
        h[1:7] = upd
    return h


def _sc_body(x_hbm, o_hbm, in_buf, out_h0, out_h1, si, so0, so1, so_e,
             *, B, L, D):
    dchunks = D // LANES
    strips_per_w = (B * dchunks) // NW
    wid = lax.axis_index("s") * NC + lax.axis_index("c")
    seg = (L - 8) // NSEG
    groups = seg // UNROLL
    halves = (out_h0, out_h1)
    sems = (so0, so1)

    def hbm_in(s):
        b, d0 = s // dchunks, (s % dchunks) * LANES
        return x_hbm.at[b, :, pl.ds(d0, LANES)]

    def hbm_out(s, r0, n):
        b, d0 = s // dchunks, (s % dchunks) * LANES
        return o_hbm.at[b, pl.ds(r0, n), pl.ds(d0, LANES)]

    first = wid * strips_per_w
    pltpu.async_copy(hbm_in(first), in_buf, si)

    def strip(k, carry):
        s = wid * strips_per_w + k
        pltpu.make_async_copy(hbm_in(s), in_buf, si).wait()

        w = tuple(in_buf[j] for j in range(8))
        h = _edge_steps(w)
        in_buf[1], in_buf[2], in_buf[3] = h[1], h[2], h[3]
        pltpu.async_copy(in_buf.at[pl.ds(0, 4)], hbm_out(s, 0, 4), so_e)

        for sg in range(NSEG):
            half, sem = halves[sg % 2], sems[sg % 2]
            if sg >= 2:
                pltpu.make_async_copy(half, hbm_out(s, 4 + sg * seg, seg),
                                      sem).wait()
            else:
                @pl.when(k > 0)
                def _():
                    pltpu.make_async_copy(half, hbm_out(s, 4 + sg * seg, seg),
                                          sem).wait()

            def group(t, w, sg=sg, half=half):
                base = 4 + sg * seg + t * UNROLL
                for u in range(UNROLL):
                    w8 = in_buf[base + u + 4]
                    out = (D0 * w[4] + D1 * (w[3] + w[5]) + D2 * (w[2] + w[6])
                           + D3 * (w[1] + w[7]) + D4 * (w[0] + w8))
                    half[t * UNROLL + u] = out
                    w = w[1:] + (w8,)
                return w

            w = lax.fori_loop(0, groups, group, w)
            pltpu.async_copy(half, hbm_out(s, 4 + sg * seg, seg), sem)

        t = _edge_steps(w)
        in_buf[L - 4], in_buf[L - 3], in_buf[L - 2] = t[4], t[5], t[6]
        pltpu.async_copy(in_buf.at[pl.ds(L - 4, 4)], hbm_out(s, L - 4, 4), so_e)
        pltpu.make_async_copy(in_buf.at[pl.ds(0, 4)], hbm_out(s, 0, 4), so_e).wait()
        pltpu.make_async_copy(in_buf.at[pl.ds(L - 4, 4)], hbm_out(s, L - 4, 4), so_e).wait()

        @pl.when(k + 1 < strips_per_w)
        def _():
            pltpu.async_copy(hbm_in(s + 1), in_buf, si)

        return carry

    lax.fori_loop(0, strips_per_w, strip, 0)
    last = wid * strips_per_w + strips_per_w - 1
    for sg in (NSEG - 2, NSEG - 1):
        pltpu.make_async_copy(halves[sg % 2], hbm_out(last, 4 + sg * seg, seg),
                              sems[sg % 2]).wait()


def _sc_mixer(x):
    B, L, D = x.shape
    assert D % LANES == 0 and (B * (D // LANES)) % NW == 0
    assert (L - 8) % NSEG == 0 and ((L - 8) // NSEG) % UNROLL == 0

    import functools
    body = functools.partial(_sc_body, B=B, L=L, D=D)
    mesh = plsc.VectorSubcoreMesh(core_axis_name="c", subcore_axis_name="s")
    return pl.kernel(
        body,
        out_type=jax.ShapeDtypeStruct((B, L, D), jnp.float32),
        mesh=mesh,
        scratch_types=[
            pltpu.VMEM((L, LANES), jnp.float32),
            pltpu.VMEM(((L - 8) // NSEG, LANES), jnp.float32),
            pltpu.VMEM(((L - 8) // NSEG, LANES), jnp.float32),
            pltpu.SemaphoreType.DMA,
            pltpu.SemaphoreType.DMA,
            pltpu.SemaphoreType.DMA,
            pltpu.SemaphoreType.DMA,
        ],
        compiler_params=pltpu.CompilerParams(use_tc_tiling_on_sc=False),
    )(x)


def _tc_block(x_ref, o_ref):
    y = x_ref[0]
    L = y.shape[0]

    def edges(h):
        for _ in range(STEPS):
            upd = h[1:7] + ALPHA * (h[2:8] - 2.0 * h[1:7] + h[0:6])
            h = jnp.concatenate([h[:1], upd, h[7:]], axis=0)
        return h

    h = edges(y[0:8])
    t = edges(y[L - 8:L])
    mid = (D0 * y[4:-4] + D1 * (y[3:-5] + y[5:-3]) + D2 * (y[2:-6] + y[6:-2])
           + D3 * (y[1:-7] + y[7:-1]) + D4 * (y[:-8] + y[8:]))
    o_ref[0] = jnp.concatenate(
        [y[:1], h[1:4], mid, t[4:7], y[-1:]], axis=0)


def _tc_mixer(x):
    B, L, D = x.shape
    W = 128
    return pl.pallas_call(
        _tc_block,
        grid=(B, D // W),
        in_specs=[pl.BlockSpec((1, L, W), lambda i, j: (i, 0, j))],
        out_specs=pl.BlockSpec((1, L, W), lambda i, j: (i, 0, j)),
        out_shape=jax.ShapeDtypeStruct((B, L, D), jnp.float32),
    )(x)


SC_BATCHES = 0


@jax.jit
def kernel(x):
    B = x.shape[0]
    sc_out = _sc_mixer(x[:SC_BATCHES])
    tc_out = _tc_mixer(x[SC_BATCHES:]) if SC_BATCHES < B else None
    if tc_out is None:
        return sc_out
    return jnp.concatenate([sc_out, tc_out], axis=0)
